# baseline (speedup 1.0000x reference)
import functools

import jax
import jax.numpy as jnp
from jax import lax
from jax.experimental import pallas as pl
from jax.experimental.pallas import tpu as pltpu
from jax.experimental.pallas import tpu_sc as plsc

N_NODES = 10000
N_EDGES = 320000
D = 128

NC = 2
NS = 16
NW = NC * NS
CHUNK = 128
NCH = 80
EDGES_PER_TILE = NCH * CHUNK
EDGES_PAD = NW * EDGES_PER_TILE
N_CHUNKS = EDGES_PAD // CHUNK
NBUF = 3
NIDX = 6
UNROLL = 6
N_MAIN = (NCH // UNROLL) * UNROLL
ROWS_A = 632
ROWS_B = N_NODES - 15 * ROWS_A


def _make_sc_scatter():
    mesh = plsc.VectorSubcoreMesh(core_axis_name="c", subcore_axis_name="s")

    @functools.partial(
        pl.kernel,
        mesh=mesh,
        out_type=jax.ShapeDtypeStruct((NC, N_NODES, D), jnp.float32),
        scratch_types=(
            [pltpu.VMEM((2, CHUNK), jnp.int32) for _ in range(NIDX)]
            + [pltpu.VMEM((CHUNK, D), jnp.float32) for _ in range(NBUF)]
            + [pltpu.VMEM_SHARED((N_NODES, D), jnp.float32)]
            + [pltpu.SemaphoreType.DMA for _ in range(2 * NBUF + NIDX)]
        ),
    )
    def sc_scatter(idx_hbm, x_hbm, zeros_hbm, out_hbm,
                   ix0, ix1, ix2, ix3, ix4, ix5,
                   rows0, rows1, rows2, h_sh,
                   gs0, gs1, gs2, ss0, ss1, ss2,
                   is0, is1, is2, is3, is4, is5):
        cid = lax.axis_index("c")
        sid = lax.axis_index("s")
        wid = sid * NC + cid

        idxs = (ix0, ix1, ix2, ix3, ix4, ix5)
        rows = (rows0, rows1, rows2)
        gsems = (gs0, gs1, gs2)
        ssems = (ss0, ss1, ss2)
        isems = (is0, is1, is2, is3, is4, is5)

        row0 = sid * ROWS_A

        @pl.when(sid < NS - 1)
        def _():
            pltpu.sync_copy(zeros_hbm.at[pl.ds(row0, ROWS_A)],
                            h_sh.at[pl.ds(row0, ROWS_A)])

        @pl.when(sid == NS - 1)
        def _():
            pltpu.sync_copy(zeros_hbm.at[pl.ds(row0, ROWS_B)],
                            h_sh.at[pl.ds(row0, ROWS_B)])

        plsc.subcore_barrier()

        chunk0 = wid * NCH

        def load_idx(j, s):
            pltpu.async_copy(idx_hbm.at[chunk0 + j], idxs[s], isems[s])

        def wait_idx(j, s):
            pltpu.make_async_copy(idx_hbm.at[chunk0 + j], idxs[s],
                                  isems[s]).wait()

        def start_gather(s, k):
            pltpu.async_copy(x_hbm.at[idxs[s].at[0]], rows[k], gsems[k])

        def wait_gather(s, k):
            pltpu.make_async_copy(x_hbm.at[idxs[s].at[0]], rows[k],
                                  gsems[k]).wait()

        def start_scatter(s, k):
            pltpu.async_copy(rows[k], h_sh.at[idxs[s].at[1]], ssems[k],
                             add=True)

        def wait_scatter(s, k):
            pltpu.make_async_copy(rows[k], h_sh.at[idxs[s].at[1]],
                                  ssems[k]).wait()

        for j in range(4):
            load_idx(j, j)
        for j in range(2):
            wait_idx(j, j)
            start_gather(j, j)

        def step(j, c, last=False):
            k = c % NBUF
            s = c % NIDX
            k2 = (c + 2) % NBUF
            s2 = (c + 2) % NIDX
            s4 = (c + 4) % NIDX
            s1 = (c + 5) % NIDX

            wait_gather(s, k)
            start_scatter(s, k)

            if c == 0 and not last:
                @pl.when(j >= 1)
                def _():
                    wait_scatter(s1, k2)
            else:
                wait_scatter(s1, k2)

            if not last:
                wait_idx(j + 2, s2)
                start_gather(s2, k2)

                @pl.when(j + 4 < NCH)
                def _():
                    load_idx(j + 4, s4)

        def ring(i, carry):
            j = UNROLL * i
            for c in range(UNROLL):
                step(j + c, c)
            return carry

        lax.fori_loop(0, N_MAIN // UNROLL, ring, 0)

        step(N_MAIN, N_MAIN % UNROLL, last=True)
        step(N_MAIN + 1, (N_MAIN + 1) % UNROLL, last=True)

        wait_scatter((NCH - 1) % NIDX, (NCH - 1) % NBUF)
        plsc.subcore_barrier()

        @pl.when(sid < NS - 1)
        def _():
            pltpu.sync_copy(h_sh.at[pl.ds(row0, ROWS_A)],
                            out_hbm.at[cid, pl.ds(row0, ROWS_A)])

        @pl.when(sid == NS - 1)
        def _():
            pltpu.sync_copy(h_sh.at[pl.ds(row0, ROWS_B)],
                            out_hbm.at[cid, pl.ds(row0, ROWS_B)])

    return sc_scatter


_sc_scatter = _make_sc_scatter()


def _tc_linear_body(h_ref, wt_ref, b_ref, o_ref):
    z = h_ref[0] + h_ref[1]
    acc = jnp.dot(z, wt_ref[...], preferred_element_type=jnp.float32)
    o_ref[...] = jnp.maximum(acc + b_ref[...], 0.0)


ROW_BLK = 1000


def _tc_linear(h, wt, b2):
    return pl.pallas_call(
        _tc_linear_body,
        grid=(N_NODES // ROW_BLK,),
        in_specs=[
            pl.BlockSpec((NC, ROW_BLK, D), lambda i: (0, i, 0)),
            pl.BlockSpec((D, D), lambda i: (0, 0)),
            pl.BlockSpec((1, D), lambda i: (0, 0)),
        ],
        out_specs=pl.BlockSpec((ROW_BLK, D), lambda i: (i, 0)),
        out_shape=jax.ShapeDtypeStruct((N_NODES, D), jnp.float32),
    )(h, wt, b2)


def kernel(inputs, edge_index, W, b):
    src = edge_index[0].astype(jnp.int32)
    dst = edge_index[1].astype(jnp.int32)
    pad = EDGES_PAD - N_EDGES
    pad_src = jnp.full((pad,), N_NODES, jnp.int32)
    pad_dst = jnp.arange(pad, dtype=jnp.int32) % N_NODES
    src_p = jnp.concatenate([src, pad_src]).reshape(N_CHUNKS, 1, CHUNK)
    dst_p = jnp.concatenate([dst, pad_dst]).reshape(N_CHUNKS, 1, CHUNK)
    idx = jnp.concatenate([src_p, dst_p], axis=1)
    x_pad = jnp.concatenate([inputs, jnp.zeros((1, D), jnp.float32)])
    zeros = jnp.zeros((N_NODES, D), jnp.float32)
    h = _sc_scatter(idx, x_pad, zeros)
    return _tc_linear(h, W.T, b.reshape(1, D))


# speedup vs baseline: 3.0624x; 3.0624x over previous
import functools

import jax
import jax.numpy as jnp
from jax import lax
from jax.experimental import pallas as pl
from jax.experimental.pallas import tpu as pltpu
from jax.experimental.pallas import tpu_sc as plsc

N_NODES = 10000
N_EDGES = 320000
D = 128

NC = 2
NS = 16
NW = NC * NS
CHUNK = 128
NCH = 80
EDGES_PER_TILE = NCH * CHUNK
EDGES_PAD = NW * EDGES_PER_TILE
N_CHUNKS = EDGES_PAD // CHUNK
NBUF = 3
NIDX = 6
UNROLL = 6
N_MAIN = (NCH // UNROLL) * UNROLL
ROWS_A = 632
ROWS_B = N_NODES - 15 * ROWS_A


def _make_sc_scatter():
    mesh = plsc.VectorSubcoreMesh(core_axis_name="c", subcore_axis_name="s")

    @functools.partial(
        pl.kernel,
        mesh=mesh,
        out_type=jax.ShapeDtypeStruct((NC, N_NODES, D), jnp.float32),
        scratch_types=(
            [pltpu.VMEM((2, CHUNK), jnp.int32) for _ in range(NIDX)]
            + [pltpu.VMEM((CHUNK, D), jnp.float32) for _ in range(NBUF)]
            + [pltpu.VMEM_SHARED((N_NODES, D), jnp.float32)]
            + [pltpu.SemaphoreType.DMA for _ in range(2 * NBUF + NIDX)]
        ),
    )
    def sc_scatter(idx_hbm, x_hbm, zeros_hbm, out_hbm,
                   ix0, ix1, ix2, ix3, ix4, ix5,
                   rows0, rows1, rows2, h_sh,
                   gs0, gs1, gs2, ss0, ss1, ss2,
                   is0, is1, is2, is3, is4, is5):
        cid = lax.axis_index("c")
        sid = lax.axis_index("s")
        wid = sid * NC + cid

        idxs = (ix0, ix1, ix2, ix3, ix4, ix5)
        rows = (rows0, rows1, rows2)
        gsems = (gs0, gs1, gs2)
        ssems = (ss0, ss1, ss2)
        isems = (is0, is1, is2, is3, is4, is5)

        row0 = sid * ROWS_A

        @pl.when(sid < NS - 1)
        def _():
            pltpu.sync_copy(zeros_hbm.at[pl.ds(row0, ROWS_A)],
                            h_sh.at[pl.ds(row0, ROWS_A)])

        @pl.when(sid == NS - 1)
        def _():
            pltpu.sync_copy(zeros_hbm.at[pl.ds(row0, ROWS_B)],
                            h_sh.at[pl.ds(row0, ROWS_B)])

        plsc.subcore_barrier()

        chunk0 = wid * NCH

        def load_idx(j, s):
            pltpu.async_copy(idx_hbm.at[chunk0 + j], idxs[s], isems[s])

        def wait_idx(j, s):
            pltpu.make_async_copy(idx_hbm.at[chunk0 + j], idxs[s],
                                  isems[s]).wait()

        def start_gather(s, k):
            pltpu.async_copy(x_hbm.at[idxs[s].at[0]], rows[k], gsems[k])

        def wait_gather(s, k):
            pltpu.make_async_copy(x_hbm.at[idxs[s].at[0]], rows[k],
                                  gsems[k]).wait()

        def start_scatter(s, k):
            pltpu.async_copy(rows[k], h_sh.at[idxs[s].at[1]], ssems[k],
                             add=True)

        def wait_scatter(s, k):
            pltpu.make_async_copy(rows[k], h_sh.at[idxs[s].at[1]],
                                  ssems[k]).wait()

        for j in range(4):
            load_idx(j, j)
        for j in range(2):
            wait_idx(j, j)
            start_gather(j, j)

        def step(j, c, last=False):
            k = c % NBUF
            s = c % NIDX
            k2 = (c + 2) % NBUF
            s2 = (c + 2) % NIDX
            s4 = (c + 4) % NIDX
            s1 = (c + 5) % NIDX

            wait_gather(s, k)
            start_scatter(s, k)

            if c == 0 and not last:
                @pl.when(j >= 1)
                def _():
                    wait_scatter(s1, k2)
            else:
                wait_scatter(s1, k2)

            if not last:
                wait_idx(j + 2, s2)
                start_gather(s2, k2)

                @pl.when(j + 4 < NCH)
                def _():
                    load_idx(j + 4, s4)

        def ring(i, carry):
            j = UNROLL * i
            for c in range(UNROLL):
                step(j + c, c)
            return carry

        lax.fori_loop(0, N_MAIN // UNROLL, ring, 0)

        step(N_MAIN, N_MAIN % UNROLL, last=True)
        step(N_MAIN + 1, (N_MAIN + 1) % UNROLL, last=True)

        wait_scatter((NCH - 1) % NIDX, (NCH - 1) % NBUF)
        plsc.subcore_barrier()

        @pl.when(sid < NS - 1)
        def _():
            pltpu.sync_copy(h_sh.at[pl.ds(row0, ROWS_A)],
                            out_hbm.at[cid, pl.ds(row0, ROWS_A)])

        @pl.when(sid == NS - 1)
        def _():
            pltpu.sync_copy(h_sh.at[pl.ds(row0, ROWS_B)],
                            out_hbm.at[cid, pl.ds(row0, ROWS_B)])

    return sc_scatter


_sc_scatter = _make_sc_scatter()


def _tc_linear_body(h_ref, wt_ref, b_ref, o_ref):
    z = h_ref[0] + h_ref[1]
    acc = jnp.dot(z, wt_ref[...], preferred_element_type=jnp.float32)
    o_ref[...] = jnp.maximum(acc + b_ref[...], 0.0)


ROW_BLK = 1000


def _tc_linear(h, wt, b2):
    return pl.pallas_call(
        _tc_linear_body,
        grid=(N_NODES // ROW_BLK,),
        in_specs=[
            pl.BlockSpec((NC, ROW_BLK, D), lambda i: (0, i, 0)),
            pl.BlockSpec((D, D), lambda i: (0, 0)),
            pl.BlockSpec((1, D), lambda i: (0, 0)),
        ],
        out_specs=pl.BlockSpec((ROW_BLK, D), lambda i: (i, 0)),
        out_shape=jax.ShapeDtypeStruct((N_NODES, D), jnp.float32),
    )(h, wt, b2)


def kernel(inputs, edge_index, W, b):
    src = edge_index[0].astype(jnp.int32)
    dst = edge_index[1].astype(jnp.int32)
    pad = EDGES_PAD - N_EDGES
    pad_src = N_NODES + (jnp.arange(pad, dtype=jnp.int32) % 128)
    pad_dst = jnp.arange(pad, dtype=jnp.int32) % N_NODES
    src_p = jnp.concatenate([src, pad_src]).reshape(N_CHUNKS, 1, CHUNK)
    dst_p = jnp.concatenate([dst, pad_dst]).reshape(N_CHUNKS, 1, CHUNK)
    idx = jnp.concatenate([src_p, dst_p], axis=1)
    x_pad = jnp.concatenate([inputs, jnp.zeros((128, D), jnp.float32)])
    zeros = jnp.zeros((N_NODES, D), jnp.float32)
    h = _sc_scatter(idx, x_pad, zeros)
    return _tc_linear(h, W.T, b.reshape(1, D))


# speedup vs baseline: 3.4243x; 1.1182x over previous
import functools

import jax
import jax.numpy as jnp
from jax import lax
from jax.experimental import pallas as pl
from jax.experimental.pallas import tpu as pltpu
from jax.experimental.pallas import tpu_sc as plsc

N_NODES = 10000
N_EDGES = 320000
D = 128

NC = 2
NS = 16
NW = NC * NS
CHUNK = 128
N_CHUNKS = N_EDGES // CHUNK
NCH_BASE = N_CHUNKS // NW
N_EXTRA = N_CHUNKS - NW * NCH_BASE
NBUF = 3
NIDX = 6
UNROLL = 6
ROWS_A = 632
ROWS_B = N_NODES - (NS - 1) * ROWS_A


def _make_sc_scatter():
    mesh = plsc.VectorSubcoreMesh(core_axis_name="c", subcore_axis_name="s")

    @functools.partial(
        pl.kernel,
        mesh=mesh,
        out_type=jax.ShapeDtypeStruct((NC, N_NODES, D), jnp.float32),
        scratch_types=(
            [pltpu.VMEM((1, CHUNK), jnp.int32) for _ in range(NIDX)]
            + [pltpu.VMEM((1, CHUNK), jnp.int32) for _ in range(NIDX)]
            + [pltpu.VMEM((CHUNK, D), jnp.float32) for _ in range(NBUF)]
            + [pltpu.VMEM_SHARED((N_NODES, D), jnp.float32)]
            + [pltpu.SemaphoreType.DMA for _ in range(2 * NBUF + NIDX)]
        ),
    )
    def sc_scatter(src_hbm, dst_hbm, x_hbm, zeros_hbm, out_hbm,
                   sa0, sa1, sa2, sa3, sa4, sa5,
                   da0, da1, da2, da3, da4, da5,
                   rows0, rows1, rows2, h_sh,
                   gs0, gs1, gs2, ss0, ss1, ss2,
                   is0, is1, is2, is3, is4, is5):
        cid = lax.axis_index("c")
        sid = lax.axis_index("s")
        wid = sid * NC + cid

        srcs = (sa0, sa1, sa2, sa3, sa4, sa5)
        dsts = (da0, da1, da2, da3, da4, da5)
        rows = (rows0, rows1, rows2)
        gsems = (gs0, gs1, gs2)
        ssems = (ss0, ss1, ss2)
        isems = (is0, is1, is2, is3, is4, is5)

        row0 = sid * ROWS_A

        @pl.when(sid < NS - 1)
        def _():
            pltpu.sync_copy(zeros_hbm.at[pl.ds(row0, ROWS_A)],
                            h_sh.at[pl.ds(row0, ROWS_A)])

        @pl.when(sid == NS - 1)
        def _():
            pltpu.sync_copy(zeros_hbm.at[pl.ds(row0, ROWS_B)],
                            h_sh.at[pl.ds(row0, ROWS_B)])

        plsc.subcore_barrier()

        has_extra = wid < N_EXTRA
        chunk0 = wid * NCH_BASE + lax.min(wid, N_EXTRA)
        nch = NCH_BASE + has_extra.astype(jnp.int32)

        def load_idx(j, s):
            pltpu.async_copy(src_hbm.at[chunk0 + j], srcs[s], isems[s])
            pltpu.async_copy(dst_hbm.at[chunk0 + j], dsts[s], isems[s])

        def wait_idx(j, s):
            pltpu.make_async_copy(src_hbm.at[chunk0 + j], srcs[s],
                                  isems[s]).wait()
            pltpu.make_async_copy(dst_hbm.at[chunk0 + j], dsts[s],
                                  isems[s]).wait()

        def start_gather(s, k):
            pltpu.async_copy(x_hbm.at[srcs[s].at[0]], rows[k], gsems[k])

        def wait_gather(s, k):
            pltpu.make_async_copy(x_hbm.at[srcs[s].at[0]], rows[k],
                                  gsems[k]).wait()

        def start_scatter(s, k):
            pltpu.async_copy(rows[k], h_sh.at[dsts[s].at[0]], ssems[k],
                             add=True)

        def wait_scatter(s, k):
            pltpu.make_async_copy(rows[k], h_sh.at[dsts[s].at[0]],
                                  ssems[k]).wait()

        for j in range(4):
            load_idx(j, j)
        for j in range(2):
            wait_idx(j, j)
            start_gather(j, j)

        def step(j, c):
            k = c % NBUF
            s = c % NIDX
            k2 = (c + 2) % NBUF
            s2 = (c + 2) % NIDX
            s4 = (c + 4) % NIDX
            s1 = (c + 5) % NIDX

            wait_gather(s, k)
            start_scatter(s, k)

            if c == 0:
                @pl.when(j >= 1)
                def _():
                    wait_scatter(s1, k2)
            else:
                wait_scatter(s1, k2)

            @pl.when(j + 2 < nch)
            def _():
                wait_idx(j + 2, s2)
                start_gather(s2, k2)

            @pl.when(j + 4 < nch)
            def _():
                load_idx(j + 4, s4)

        def ring(i, carry):
            j = UNROLL * i
            for c in range(UNROLL):
                step(j + c, c)
            return carry

        lax.fori_loop(0, NCH_BASE // UNROLL, ring, 0)

        kt = NCH_BASE % NBUF
        st = NCH_BASE % NIDX
        s_prev = (NCH_BASE - 1) % NIDX
        k_prev = (NCH_BASE - 1) % NBUF

        @pl.when(has_extra)
        def _():
            wait_gather(st, kt)
            start_scatter(st, kt)
            wait_scatter(s_prev, k_prev)
            wait_scatter(st, kt)

        @pl.when(jnp.logical_not(has_extra))
        def _():
            wait_scatter(s_prev, k_prev)

        plsc.subcore_barrier()

        @pl.when(sid < NS - 1)
        def _():
            pltpu.sync_copy(h_sh.at[pl.ds(row0, ROWS_A)],
                            out_hbm.at[cid, pl.ds(row0, ROWS_A)])

        @pl.when(sid == NS - 1)
        def _():
            pltpu.sync_copy(h_sh.at[pl.ds(row0, ROWS_B)],
                            out_hbm.at[cid, pl.ds(row0, ROWS_B)])

    return sc_scatter


_sc_scatter = _make_sc_scatter()


def _tc_linear_body(h_ref, wt_ref, b_ref, o_ref):
    z = h_ref[0] + h_ref[1]
    acc = jnp.dot(z, wt_ref[...], preferred_element_type=jnp.float32)
    o_ref[...] = jnp.maximum(acc + b_ref[...], 0.0)


ROW_BLK = 1000


def _tc_linear(h, wt, b2):
    return pl.pallas_call(
        _tc_linear_body,
        grid=(N_NODES // ROW_BLK,),
        in_specs=[
            pl.BlockSpec((NC, ROW_BLK, D), lambda i: (0, i, 0)),
            pl.BlockSpec((D, D), lambda i: (0, 0)),
            pl.BlockSpec((1, D), lambda i: (0, 0)),
        ],
        out_specs=pl.BlockSpec((ROW_BLK, D), lambda i: (i, 0)),
        out_shape=jax.ShapeDtypeStruct((N_NODES, D), jnp.float32),
    )(h, wt, b2)


def kernel(inputs, edge_index, W, b):
    src3 = edge_index[0].astype(jnp.int32).reshape(N_CHUNKS, 1, CHUNK)
    dst3 = edge_index[1].astype(jnp.int32).reshape(N_CHUNKS, 1, CHUNK)
    zeros = jnp.zeros((N_NODES, D), jnp.float32)
    h = _sc_scatter(src3, dst3, inputs, zeros)
    return _tc_linear(h, W.T, b.reshape(1, D))


# speedup vs baseline: 3.4271x; 1.0008x over previous
import functools

import jax
import jax.numpy as jnp
from jax import lax
from jax.experimental import pallas as pl
from jax.experimental.pallas import tpu as pltpu
from jax.experimental.pallas import tpu_sc as plsc

N_NODES = 10000
N_EDGES = 320000
D = 128

NC = 2
NS = 16
NW = NC * NS
CHUNK = 128
N_CHUNKS = N_EDGES // CHUNK
NCH_BASE = N_CHUNKS // NW
N_EXTRA = N_CHUNKS - NW * NCH_BASE
NBUF = 3
NGRP = 3
UNROLL = 6
ROWS_A = 632
ROWS_B = N_NODES - (NS - 1) * ROWS_A


def _make_sc_scatter():
    mesh = plsc.VectorSubcoreMesh(core_axis_name="c", subcore_axis_name="s")

    @functools.partial(
        pl.kernel,
        mesh=mesh,
        out_type=jax.ShapeDtypeStruct((NC, N_NODES, D), jnp.float32),
        scratch_types=(
            [pltpu.VMEM((2, 1, CHUNK), jnp.int32) for _ in range(NGRP)]
            + [pltpu.VMEM((2, 1, CHUNK), jnp.int32) for _ in range(NGRP)]
            + [pltpu.VMEM((CHUNK, D), jnp.float32) for _ in range(NBUF)]
            + [pltpu.VMEM_SHARED((N_NODES, D), jnp.float32)]
            + [pltpu.SemaphoreType.DMA for _ in range(2 * NBUF + NGRP)]
        ),
    )
    def sc_scatter(src_hbm, dst_hbm, x_hbm, zeros_hbm, out_hbm,
                   sa0, sa1, sa2, da0, da1, da2,
                   rows0, rows1, rows2, h_sh,
                   gs0, gs1, gs2, ss0, ss1, ss2,
                   is0, is1, is2):
        cid = lax.axis_index("c")
        sid = lax.axis_index("s")
        wid = sid * NC + cid

        srcs = (sa0, sa1, sa2)
        dsts = (da0, da1, da2)
        rows = (rows0, rows1, rows2)
        gsems = (gs0, gs1, gs2)
        ssems = (ss0, ss1, ss2)
        isems = (is0, is1, is2)

        row0 = sid * ROWS_A

        @pl.when(sid < NS - 1)
        def _():
            pltpu.sync_copy(zeros_hbm.at[pl.ds(row0, ROWS_A)],
                            h_sh.at[pl.ds(row0, ROWS_A)])

        @pl.when(sid == NS - 1)
        def _():
            pltpu.sync_copy(zeros_hbm.at[pl.ds(row0, ROWS_B)],
                            h_sh.at[pl.ds(row0, ROWS_B)])

        plsc.subcore_barrier()

        has_extra = wid < N_EXTRA
        chunk0 = wid * NCH_BASE + lax.min(wid, N_EXTRA)
        nch = NCH_BASE + has_extra.astype(jnp.int32)

        def load_grp(g, s):
            base = chunk0 + 2 * g
            pltpu.async_copy(src_hbm.at[pl.ds(base, 2)], srcs[s], isems[s])
            pltpu.async_copy(dst_hbm.at[pl.ds(base, 2)], dsts[s], isems[s])

        def wait_grp(g, s):
            base = chunk0 + 2 * g
            pltpu.make_async_copy(src_hbm.at[pl.ds(base, 2)], srcs[s],
                                  isems[s]).wait()
            pltpu.make_async_copy(dst_hbm.at[pl.ds(base, 2)], dsts[s],
                                  isems[s]).wait()

        def start_gather(s, p, k):
            pltpu.async_copy(x_hbm.at[srcs[s].at[p, 0]], rows[k], gsems[k])

        def wait_gather(s, p, k):
            pltpu.make_async_copy(x_hbm.at[srcs[s].at[p, 0]], rows[k],
                                  gsems[k]).wait()

        def start_scatter(s, p, k):
            pltpu.async_copy(rows[k], h_sh.at[dsts[s].at[p, 0]], ssems[k],
                             add=True)

        def wait_scatter(s, p, k):
            pltpu.make_async_copy(rows[k], h_sh.at[dsts[s].at[p, 0]],
                                  ssems[k]).wait()

        for g in range(3):
            load_grp(g, g)
        wait_grp(0, 0)
        start_gather(0, 0, 0)
        start_gather(0, 1, 1)

        def step(j, c):
            k = c % NBUF
            s = (c // 2) % NGRP
            p = c % 2
            k2 = (c + 2) % NBUF
            s2 = ((c + 2) // 2) % NGRP
            p2 = (c + 2) % 2
            s1 = ((c + 5) // 2) % NGRP
            p1 = (c + 5) % 2

            wait_gather(s, p, k)
            start_scatter(s, p, k)

            if c == 0:
                @pl.when(j >= 1)
                def _():
                    wait_scatter(s1, p1, k2)
            else:
                wait_scatter(s1, p1, k2)

            @pl.when(j + 2 < nch)
            def _():
                if p2 == 0:
                    wait_grp((j + 2) // 2, s2)
                start_gather(s2, p2, k2)

            if p == 0:
                @pl.when((j >= 2) & (j + 4 < nch))
                def _():
                    load_grp(j // 2 + 2, (c // 2 + 2) % NGRP)

        def ring(i, carry):
            j = UNROLL * i
            for c in range(UNROLL):
                step(j + c, c)
            return carry

        lax.fori_loop(0, NCH_BASE // UNROLL, ring, 0)

        kt = NCH_BASE % NBUF
        st = (NCH_BASE // 2) % NGRP
        pt = NCH_BASE % 2
        s_prev = ((NCH_BASE - 1) // 2) % NGRP
        p_prev = (NCH_BASE - 1) % 2
        k_prev = (NCH_BASE - 1) % NBUF

        @pl.when(has_extra)
        def _():
            wait_gather(st, pt, kt)
            start_scatter(st, pt, kt)
            wait_scatter(s_prev, p_prev, k_prev)
            wait_scatter(st, pt, kt)

        @pl.when(jnp.logical_not(has_extra))
        def _():
            wait_scatter(s_prev, p_prev, k_prev)

        plsc.subcore_barrier()

        @pl.when(sid < NS - 1)
        def _():
            pltpu.sync_copy(h_sh.at[pl.ds(row0, ROWS_A)],
                            out_hbm.at[cid, pl.ds(row0, ROWS_A)])

        @pl.when(sid == NS - 1)
        def _():
            pltpu.sync_copy(h_sh.at[pl.ds(row0, ROWS_B)],
                            out_hbm.at[cid, pl.ds(row0, ROWS_B)])

    return sc_scatter


_sc_scatter = _make_sc_scatter()


def _tc_linear_body(h_ref, wt_ref, b_ref, o_ref):
    z = h_ref[0] + h_ref[1]
    acc = jnp.dot(z, wt_ref[...], preferred_element_type=jnp.float32)
    o_ref[...] = jnp.maximum(acc + b_ref[...], 0.0)


ROW_BLK = 1000


def _tc_linear(h, wt, b2):
    return pl.pallas_call(
        _tc_linear_body,
        grid=(N_NODES // ROW_BLK,),
        in_specs=[
            pl.BlockSpec((NC, ROW_BLK, D), lambda i: (0, i, 0)),
            pl.BlockSpec((D, D), lambda i: (0, 0)),
            pl.BlockSpec((1, D), lambda i: (0, 0)),
        ],
        out_specs=pl.BlockSpec((ROW_BLK, D), lambda i: (i, 0)),
        out_shape=jax.ShapeDtypeStruct((N_NODES, D), jnp.float32),
    )(h, wt, b2)


def kernel(inputs, edge_index, W, b):
    src3 = edge_index[0].astype(jnp.int32).reshape(N_CHUNKS, 1, CHUNK)
    dst3 = edge_index[1].astype(jnp.int32).reshape(N_CHUNKS, 1, CHUNK)
    zeros = jnp.zeros((N_NODES, D), jnp.float32)
    h = _sc_scatter(src3, dst3, inputs, zeros)
    return _tc_linear(h, W.T, b.reshape(1, D))
